# Initial kernel scaffold; baseline (speedup 1.0000x reference)
import functools
import math

import jax
import jax.numpy as jnp
from jax import lax
from jax.experimental import pallas as pl
from jax.experimental.pallas import tpu as pltpu
from jax.experimental.pallas import tpu_sc as plsc

N = 10000
R = 500
E = 320000
D = 128

NC = 2
NS = 16
NW = NC * NS
EPW = E // NW
B = 80
NB = EPW // B
G = B // 16

INV3SQD = 1.0 / (3.0 * math.sqrt(D))


def _relw_body(rs_ref, rt_ref, wsr_ref, wtg_ref):
    a = rs_ref[...]
    b = rt_ref[...]
    an = a / (jnp.sqrt(jnp.sum(a * a, axis=1, keepdims=True)) + 1e-8)
    bn = b / (jnp.sqrt(jnp.sum(b * b, axis=1, keepdims=True)) + 1e-8)
    sim = lax.dot_general(an, bn, (((1,), (1,)), ((), ())),
                          preferred_element_type=jnp.float32)
    wsr_ref[...] = jnp.max(sim, axis=1)
    wtg_ref[...] = jnp.max(sim, axis=0)


def _relation_weights(rel_emb_sr, rel_emb_tg):
    return pl.pallas_call(
        _relw_body,
        out_shape=[jax.ShapeDtypeStruct((R,), jnp.float32),
                   jax.ShapeDtypeStruct((R,), jnp.float32)],
    )(rel_emb_sr, rel_emb_tg)


def _side_phase(ent_hbm, rel_hbm, w_hbm, head_hbm, tail_hbm, rid_hbm,
                conf_hbm, imp_hbm, pca_hbm, out_hbm,
                relt_v, w_v, hidx, tidx, ridx, hrows, trows,
                confv, impv, pcav, evv, sem1, sem2, base0):
    pltpu.sync_copy(rel_hbm, relt_v)
    pltpu.sync_copy(w_hbm, w_v)

    def block(i, carry):
        base = base0 + i * B
        pltpu.sync_copy(head_hbm.at[pl.ds(base, B)], hidx)
        pltpu.sync_copy(tail_hbm.at[pl.ds(base, B)], tidx)
        pltpu.sync_copy(rid_hbm.at[pl.ds(base, B)], ridx)
        pltpu.sync_copy(conf_hbm.at[pl.ds(base, B)], confv)
        pltpu.sync_copy(imp_hbm.at[pl.ds(base, B)], impv)
        pltpu.sync_copy(pca_hbm.at[pl.ds(base, B)], pcav)
        cp1 = pltpu.async_copy(ent_hbm.at[hidx], hrows, sem1)
        cp2 = pltpu.async_copy(ent_hbm.at[tidx], trows, sem2)
        cp1.wait()
        cp2.wait()
        for g in range(G):
            rows = lax.iota(jnp.int32, 16) + (g * 16)
            rids = ridx[pl.ds(g * 16, 16)]

            def dbody(dd, acc):
                col = jnp.full((16,), dd, dtype=jnp.int32)
                hv = plsc.load_gather(hrows, [rows, col])
                tv = plsc.load_gather(trows, [rows, col])
                rv = plsc.load_gather(relt_v, [rids, col])
                return acc + jnp.abs(hv + rv - tv)

            acc = lax.fori_loop(0, D, dbody, jnp.zeros((16,), jnp.float32),
                                unroll=8)
            att = plsc.load_gather(w_v, [rids])
            tv_score = 1.0 - acc * INV3SQD
            cv = confv[pl.ds(g * 16, 16)]
            iv = impv[pl.ds(g * 16, 16)]
            pv = pcav[pl.ds(g * 16, 16)]
            evv[pl.ds(g * 16, 16)] = cv * iv * (
                0.3 * pv + 0.3 * tv_score + 0.4 * att)
        pltpu.sync_copy(evv, out_hbm.at[pl.ds(base, B)])
        return carry

    lax.fori_loop(0, NB, block, 0)


def _sc_body(ent_sr, ent_tg, rel_sr, rel_tg, w_sr, w_tg,
             head_sr, tail_sr, rid_sr, head_tg, tail_tg, rid_tg,
             conf_sr, imp_sr, pca_sr, conf_tg, imp_tg, pca_tg,
             out_sr, out_tg,
             relt_v, w_v, hidx, tidx, ridx, hrows, trows,
             confv, impv, pcav, evv, sem1, sem2):
    wid = lax.axis_index("s") * NC + lax.axis_index("c")
    base0 = wid * EPW
    _side_phase(ent_sr, rel_sr, w_sr, head_sr, tail_sr, rid_sr,
                conf_sr, imp_sr, pca_sr, out_sr,
                relt_v, w_v, hidx, tidx, ridx, hrows, trows,
                confv, impv, pcav, evv, sem1, sem2, base0)
    _side_phase(ent_tg, rel_tg, w_tg, head_tg, tail_tg, rid_tg,
                conf_tg, imp_tg, pca_tg, out_tg,
                relt_v, w_v, hidx, tidx, ridx, hrows, trows,
                confv, impv, pcav, evv, sem1, sem2, base0)


_edge_values = functools.partial(
    pl.kernel,
    out_type=[jax.ShapeDtypeStruct((E,), jnp.float32),
              jax.ShapeDtypeStruct((E,), jnp.float32)],
    mesh=plsc.VectorSubcoreMesh(core_axis_name="c", subcore_axis_name="s"),
    scratch_types=[
        pltpu.VMEM((R, D), jnp.float32),
        pltpu.VMEM((R,), jnp.float32),
        pltpu.VMEM((B,), jnp.int32),
        pltpu.VMEM((B,), jnp.int32),
        pltpu.VMEM((B,), jnp.int32),
        pltpu.VMEM((B, D), jnp.float32),
        pltpu.VMEM((B, D), jnp.float32),
        pltpu.VMEM((B,), jnp.float32),
        pltpu.VMEM((B,), jnp.float32),
        pltpu.VMEM((B,), jnp.float32),
        pltpu.VMEM((B,), jnp.float32),
        pltpu.SemaphoreType.DMA,
        pltpu.SemaphoreType.DMA,
    ],
)(_sc_body)


def kernel(ent_emb_sr, ent_emb_tg, rel_emb_sr, rel_emb_tg,
           head_sr, tail_sr, relation_sr, head_tg, tail_tg, relation_tg,
           rel_conf_sr, rel_imp_sr, pca_sr, rel_conf_tg, rel_imp_tg, pca_tg):
    w_sr, w_tg = _relation_weights(rel_emb_sr, rel_emb_tg)
    i32 = jnp.int32
    ev_sr, ev_tg = _edge_values(
        ent_emb_sr, ent_emb_tg, rel_emb_sr, rel_emb_tg, w_sr, w_tg,
        head_sr.astype(i32), tail_sr.astype(i32), relation_sr.astype(i32),
        head_tg.astype(i32), tail_tg.astype(i32), relation_tg.astype(i32),
        rel_conf_sr, rel_imp_sr, pca_sr, rel_conf_tg, rel_imp_tg, pca_tg)
    ones = jnp.ones((N,), dtype=jnp.float32)
    return (jnp.concatenate([ev_sr, ones]),
            jnp.concatenate([ev_tg, ones]))


# speedup vs baseline: 2.1543x; 2.1543x over previous
import functools
import math

import jax
import jax.numpy as jnp
from jax import lax
from jax.experimental import pallas as pl
from jax.experimental.pallas import tpu as pltpu
from jax.experimental.pallas import tpu_sc as plsc

N = 10000
R = 500
E = 320000
D = 128

NC = 2
NS = 16
NW = NC * NS
EPW = E // NW
B = 80
NB = EPW // B
G = B // 16

INV3SQD = 1.0 / (3.0 * math.sqrt(D))


def _relw_body(rs_ref, rt_ref, wsr_ref, wtg_ref):
    a = rs_ref[...]
    b = rt_ref[...]
    an = a / (jnp.sqrt(jnp.sum(a * a, axis=1, keepdims=True)) + 1e-8)
    bn = b / (jnp.sqrt(jnp.sum(b * b, axis=1, keepdims=True)) + 1e-8)
    sim = lax.dot_general(an, bn, (((1,), (1,)), ((), ())),
                          preferred_element_type=jnp.float32)
    wsr_ref[...] = jnp.max(sim, axis=1)
    wtg_ref[...] = jnp.max(sim, axis=0)


def _relation_weights(rel_emb_sr, rel_emb_tg):
    return pl.pallas_call(
        _relw_body,
        out_shape=[jax.ShapeDtypeStruct((R,), jnp.float32),
                   jax.ShapeDtypeStruct((R,), jnp.float32)],
    )(rel_emb_sr, rel_emb_tg)


def _side_phase(ent_hbm, rel_hbm, w_hbm, head_hbm, tail_hbm, rid_hbm,
                conf_hbm, imp_hbm, pca_hbm, out_hbm,
                relt_v, w_v, hidx, tidx, ridx, hrows, trows,
                confv, impv, pcav, evv, sem1, sem2, base0):
    pltpu.sync_copy(rel_hbm, relt_v)
    pltpu.sync_copy(w_hbm, w_v)

    def block(i, carry):
        base = base0 + i * B
        pltpu.sync_copy(head_hbm.at[pl.ds(base, B)], hidx)
        pltpu.sync_copy(tail_hbm.at[pl.ds(base, B)], tidx)
        pltpu.sync_copy(rid_hbm.at[pl.ds(base, B)], ridx)
        pltpu.sync_copy(conf_hbm.at[pl.ds(base, B)], confv)
        pltpu.sync_copy(imp_hbm.at[pl.ds(base, B)], impv)
        pltpu.sync_copy(pca_hbm.at[pl.ds(base, B)], pcav)
        cp1 = pltpu.async_copy(ent_hbm.at[hidx], hrows, sem1)
        cp2 = pltpu.async_copy(ent_hbm.at[tidx], trows, sem2)
        cp1.wait()
        cp2.wait()
        for g in range(G):
            rows = lax.iota(jnp.int32, 16) + (g * 16)
            rids = ridx[pl.ds(g * 16, 16)]

            def dbody(dd, acc):
                col = jnp.full((16,), dd, dtype=jnp.int32)
                hv = plsc.load_gather(hrows, [rows, col])
                tv = plsc.load_gather(trows, [rows, col])
                rv = plsc.load_gather(relt_v, [rids, col])
                return acc + jnp.abs(hv + rv - tv)

            acc = lax.fori_loop(0, D, dbody, jnp.zeros((16,), jnp.float32),
                                unroll=8)
            att = plsc.load_gather(w_v, [rids])
            tv_score = 1.0 - acc * INV3SQD
            cv = confv[pl.ds(g * 16, 16)]
            iv = impv[pl.ds(g * 16, 16)]
            pv = pcav[pl.ds(g * 16, 16)]
            evv[pl.ds(g * 16, 16)] = cv * iv * (
                0.3 * pv + 0.3 * tv_score + 0.4 * att)
        pltpu.sync_copy(evv, out_hbm.at[pl.ds(base, B)])
        return carry

    lax.fori_loop(0, NB, block, 0)


def _sc_body(ent_sr, ent_tg, rel_sr, rel_tg, w_sr, w_tg,
             head_sr, tail_sr, rid_sr, head_tg, tail_tg, rid_tg,
             conf_sr, imp_sr, pca_sr, conf_tg, imp_tg, pca_tg,
             out_sr, out_tg,
             relt_v, w_v, hidx, tidx, ridx, hrows, trows,
             confv, impv, pcav, evv, sem1, sem2):
    wid = lax.axis_index("s") * NC + lax.axis_index("c")
    base0 = wid * EPW
    _side_phase(ent_sr, rel_sr, w_sr, head_sr, tail_sr, rid_sr,
                conf_sr, imp_sr, pca_sr, out_sr,
                relt_v, w_v, hidx, tidx, ridx, hrows, trows,
                confv, impv, pcav, evv, sem1, sem2, base0)
    _side_phase(ent_tg, rel_tg, w_tg, head_tg, tail_tg, rid_tg,
                conf_tg, imp_tg, pca_tg, out_tg,
                relt_v, w_v, hidx, tidx, ridx, hrows, trows,
                confv, impv, pcav, evv, sem1, sem2, base0)


_edge_values = functools.partial(
    pl.kernel,
    out_type=[jax.ShapeDtypeStruct((E,), jnp.float32),
              jax.ShapeDtypeStruct((E,), jnp.float32)],
    mesh=plsc.VectorSubcoreMesh(core_axis_name="c", subcore_axis_name="s"),
    compiler_params=pltpu.CompilerParams(needs_layout_passes=False),
    scratch_types=[
        pltpu.VMEM((R, D), jnp.float32),
        pltpu.VMEM((R,), jnp.float32),
        pltpu.VMEM((B,), jnp.int32),
        pltpu.VMEM((B,), jnp.int32),
        pltpu.VMEM((B,), jnp.int32),
        pltpu.VMEM((B, D), jnp.float32),
        pltpu.VMEM((B, D), jnp.float32),
        pltpu.VMEM((B,), jnp.float32),
        pltpu.VMEM((B,), jnp.float32),
        pltpu.VMEM((B,), jnp.float32),
        pltpu.VMEM((B,), jnp.float32),
        pltpu.SemaphoreType.DMA,
        pltpu.SemaphoreType.DMA,
    ],
)(_sc_body)


def kernel(ent_emb_sr, ent_emb_tg, rel_emb_sr, rel_emb_tg,
           head_sr, tail_sr, relation_sr, head_tg, tail_tg, relation_tg,
           rel_conf_sr, rel_imp_sr, pca_sr, rel_conf_tg, rel_imp_tg, pca_tg):
    w_sr, w_tg = _relation_weights(rel_emb_sr, rel_emb_tg)
    i32 = jnp.int32
    ev_sr, ev_tg = _edge_values(
        ent_emb_sr, ent_emb_tg, rel_emb_sr, rel_emb_tg, w_sr, w_tg,
        head_sr.astype(i32), tail_sr.astype(i32), relation_sr.astype(i32),
        head_tg.astype(i32), tail_tg.astype(i32), relation_tg.astype(i32),
        rel_conf_sr, rel_imp_sr, pca_sr, rel_conf_tg, rel_imp_tg, pca_tg)
    ones = jnp.ones((N,), dtype=jnp.float32)
    return (jnp.concatenate([ev_sr, ones]),
            jnp.concatenate([ev_tg, ones]))


# speedup vs baseline: 7.3312x; 3.4031x over previous
import functools
import math

import jax
import jax.numpy as jnp
from jax import lax
from jax.experimental import pallas as pl
from jax.experimental.pallas import tpu as pltpu
from jax.experimental.pallas import tpu_sc as plsc

N = 10000
R = 500
E = 320000
D = 128

NC = 2
NS = 16
NW = NC * NS
EPW = E // NW
B = 80
NB = EPW // B
G = B // 16

INV3SQD = 1.0 / (3.0 * math.sqrt(D))


def _relw_body(rs_ref, rt_ref, wsr_ref, wtg_ref):
    a = rs_ref[...]
    b = rt_ref[...]
    an = a / (jnp.sqrt(jnp.sum(a * a, axis=1, keepdims=True)) + 1e-8)
    bn = b / (jnp.sqrt(jnp.sum(b * b, axis=1, keepdims=True)) + 1e-8)
    sim = lax.dot_general(an, bn, (((1,), (1,)), ((), ())),
                          preferred_element_type=jnp.float32)
    wsr_ref[...] = jnp.max(sim, axis=1)
    wtg_ref[...] = jnp.max(sim, axis=0)


def _relation_weights(rel_emb_sr, rel_emb_tg):
    return pl.pallas_call(
        _relw_body,
        out_shape=[jax.ShapeDtypeStruct((R,), jnp.float32),
                   jax.ShapeDtypeStruct((R,), jnp.float32)],
    )(rel_emb_sr, rel_emb_tg)


def _side_phase(ent_hbm, rel_hbm, w_hbm, head_hbm, tail_hbm, rid_hbm,
                conf_hbm, imp_hbm, pca_hbm, out_hbm,
                relt_v, w_v, hidx, tidx, ridx, hrows, trows,
                confv, impv, pcav, evv, sem1, sem2, base0):
    pltpu.sync_copy(rel_hbm, relt_v)
    pltpu.sync_copy(w_hbm, w_v)

    def block(i, carry):
        base = base0 + i * B
        pltpu.sync_copy(head_hbm.at[pl.ds(base, B)], hidx)
        pltpu.sync_copy(tail_hbm.at[pl.ds(base, B)], tidx)
        pltpu.sync_copy(rid_hbm.at[pl.ds(base, B)], ridx)
        pltpu.sync_copy(conf_hbm.at[pl.ds(base, B)], confv)
        pltpu.sync_copy(imp_hbm.at[pl.ds(base, B)], impv)
        pltpu.sync_copy(pca_hbm.at[pl.ds(base, B)], pcav)
        cp1 = pltpu.async_copy(ent_hbm.at[hidx], hrows, sem1)
        cp2 = pltpu.async_copy(ent_hbm.at[tidx], trows, sem2)
        cp1.wait()
        cp2.wait()
        for g in range(G):
            lane = lax.iota(jnp.int32, 16)
            rows = lane + (g * 16)
            rids = ridx[pl.ds(g * 16, 16)]

            def dbody(dd, acc):
                col = (lane + dd) & (D - 1)
                hv = plsc.load_gather(hrows, [rows, col])
                tv = plsc.load_gather(trows, [rows, col])
                rv = plsc.load_gather(relt_v, [rids, col])
                return acc + jnp.abs(hv + rv - tv)

            acc = lax.fori_loop(0, D, dbody, jnp.zeros((16,), jnp.float32),
                                unroll=8)
            att = plsc.load_gather(w_v, [rids])
            tv_score = 1.0 - acc * INV3SQD
            cv = confv[pl.ds(g * 16, 16)]
            iv = impv[pl.ds(g * 16, 16)]
            pv = pcav[pl.ds(g * 16, 16)]
            evv[pl.ds(g * 16, 16)] = cv * iv * (
                0.3 * pv + 0.3 * tv_score + 0.4 * att)
        pltpu.sync_copy(evv, out_hbm.at[pl.ds(base, B)])
        return carry

    lax.fori_loop(0, NB, block, 0)


def _sc_body(ent_sr, ent_tg, rel_sr, rel_tg, w_sr, w_tg,
             head_sr, tail_sr, rid_sr, head_tg, tail_tg, rid_tg,
             conf_sr, imp_sr, pca_sr, conf_tg, imp_tg, pca_tg,
             out_sr, out_tg,
             relt_v, w_v, hidx, tidx, ridx, hrows, trows,
             confv, impv, pcav, evv, sem1, sem2):
    wid = lax.axis_index("s") * NC + lax.axis_index("c")
    base0 = wid * EPW
    _side_phase(ent_sr, rel_sr, w_sr, head_sr, tail_sr, rid_sr,
                conf_sr, imp_sr, pca_sr, out_sr,
                relt_v, w_v, hidx, tidx, ridx, hrows, trows,
                confv, impv, pcav, evv, sem1, sem2, base0)
    _side_phase(ent_tg, rel_tg, w_tg, head_tg, tail_tg, rid_tg,
                conf_tg, imp_tg, pca_tg, out_tg,
                relt_v, w_v, hidx, tidx, ridx, hrows, trows,
                confv, impv, pcav, evv, sem1, sem2, base0)


_edge_values = functools.partial(
    pl.kernel,
    out_type=[jax.ShapeDtypeStruct((E,), jnp.float32),
              jax.ShapeDtypeStruct((E,), jnp.float32)],
    mesh=plsc.VectorSubcoreMesh(core_axis_name="c", subcore_axis_name="s"),
    compiler_params=pltpu.CompilerParams(needs_layout_passes=False),
    scratch_types=[
        pltpu.VMEM((R, D), jnp.float32),
        pltpu.VMEM((R,), jnp.float32),
        pltpu.VMEM((B,), jnp.int32),
        pltpu.VMEM((B,), jnp.int32),
        pltpu.VMEM((B,), jnp.int32),
        pltpu.VMEM((B, D), jnp.float32),
        pltpu.VMEM((B, D), jnp.float32),
        pltpu.VMEM((B,), jnp.float32),
        pltpu.VMEM((B,), jnp.float32),
        pltpu.VMEM((B,), jnp.float32),
        pltpu.VMEM((B,), jnp.float32),
        pltpu.SemaphoreType.DMA,
        pltpu.SemaphoreType.DMA,
    ],
)(_sc_body)


def kernel(ent_emb_sr, ent_emb_tg, rel_emb_sr, rel_emb_tg,
           head_sr, tail_sr, relation_sr, head_tg, tail_tg, relation_tg,
           rel_conf_sr, rel_imp_sr, pca_sr, rel_conf_tg, rel_imp_tg, pca_tg):
    w_sr, w_tg = _relation_weights(rel_emb_sr, rel_emb_tg)
    i32 = jnp.int32
    ev_sr, ev_tg = _edge_values(
        ent_emb_sr, ent_emb_tg, rel_emb_sr, rel_emb_tg, w_sr, w_tg,
        head_sr.astype(i32), tail_sr.astype(i32), relation_sr.astype(i32),
        head_tg.astype(i32), tail_tg.astype(i32), relation_tg.astype(i32),
        rel_conf_sr, rel_imp_sr, pca_sr, rel_conf_tg, rel_imp_tg, pca_tg)
    ones = jnp.ones((N,), dtype=jnp.float32)
    return (jnp.concatenate([ev_sr, ones]),
            jnp.concatenate([ev_tg, ones]))


# speedup vs baseline: 21.6414x; 2.9519x over previous
import functools
import math

import jax
import jax.numpy as jnp
from jax import lax
from jax.experimental import pallas as pl
from jax.experimental.pallas import tpu as pltpu
from jax.experimental.pallas import tpu_sc as plsc

N = 10000
R = 500
E = 320000
D = 128

NC = 2
NS = 16
NW = NC * NS
EPW = E // NW
B = 80
G = B // 16
SB = 2000
NSB = EPW // SB
BPS = SB // B

INV3SQD = 1.0 / (3.0 * math.sqrt(D))


def _relw_body(rs_ref, rt_ref, wsr_ref, wtg_ref):
    a = rs_ref[...]
    b = rt_ref[...]
    an = a / (jnp.sqrt(jnp.sum(a * a, axis=1, keepdims=True)) + 1e-8)
    bn = b / (jnp.sqrt(jnp.sum(b * b, axis=1, keepdims=True)) + 1e-8)
    sim = lax.dot_general(an, bn, (((1,), (1,)), ((), ())),
                          preferred_element_type=jnp.float32)
    wsr_ref[...] = jnp.max(sim, axis=1)
    wtg_ref[...] = jnp.max(sim, axis=0)


def _relation_weights(rel_emb_sr, rel_emb_tg):
    return pl.pallas_call(
        _relw_body,
        out_shape=[jax.ShapeDtypeStruct((R,), jnp.float32),
                   jax.ShapeDtypeStruct((R,), jnp.float32)],
    )(rel_emb_sr, rel_emb_tg)


def _compute_block(hrows, trows, relt_v, w_v, ridx, confv, impv, pcav,
                   evv, boff):
    for g in range(G):
        lane = lax.iota(jnp.int32, 16)
        rows = lane + (g * 16)
        off = boff + g * 16
        rids = ridx[pl.ds(off, 16)]

        def dbody(dd, acc):
            col = (lane + dd) & (D - 1)
            hv = plsc.load_gather(hrows, [rows, col])
            tv = plsc.load_gather(trows, [rows, col])
            rv = plsc.load_gather(relt_v, [rids, col])
            return acc + jnp.abs(hv + rv - tv)

        acc = lax.fori_loop(0, D, dbody, jnp.zeros((16,), jnp.float32),
                            unroll=8)
        att = plsc.load_gather(w_v, [rids])
        tv_score = 1.0 - acc * INV3SQD
        cv = confv[pl.ds(off, 16)]
        iv = impv[pl.ds(off, 16)]
        pv = pcav[pl.ds(off, 16)]
        evv[pl.ds(off, 16)] = cv * iv * (0.3 * pv + 0.3 * tv_score + 0.4 * att)


def _side_phase(ent_hbm, rel_hbm, w_hbm, head_hbm, tail_hbm, rid_hbm,
                conf_hbm, imp_hbm, pca_hbm, out_hbm,
                relt_v, w_v, hidx, tidx, ridx,
                hrA, trA, hrB, trB,
                confv, impv, pcav, evv, semA, semB, base0):
    pltpu.sync_copy(rel_hbm, relt_v)
    pltpu.sync_copy(w_hbm, w_v)

    def issue(bi, hr, tr, sem):
        off = bi * B
        pltpu.async_copy(ent_hbm.at[hidx.at[pl.ds(off, B)]], hr, sem)
        pltpu.async_copy(ent_hbm.at[tidx.at[pl.ds(off, B)]], tr, sem)

    def drain(hr, tr, sem):
        pltpu.make_async_copy(ent_hbm.at[hidx.at[pl.ds(0, B)]], hr, sem).wait()
        pltpu.make_async_copy(ent_hbm.at[tidx.at[pl.ds(0, B)]], tr, sem).wait()

    def superblock(sbi, carry):
        sbbase = base0 + sbi * SB
        pltpu.sync_copy(head_hbm.at[pl.ds(sbbase, SB)], hidx)
        pltpu.sync_copy(tail_hbm.at[pl.ds(sbbase, SB)], tidx)
        pltpu.sync_copy(rid_hbm.at[pl.ds(sbbase, SB)], ridx)
        pltpu.sync_copy(conf_hbm.at[pl.ds(sbbase, SB)], confv)
        pltpu.sync_copy(imp_hbm.at[pl.ds(sbbase, SB)], impv)
        pltpu.sync_copy(pca_hbm.at[pl.ds(sbbase, SB)], pcav)
        issue(0, hrA, trA, semA)

        def pair(i, c):
            b = 2 * i
            issue(b + 1, hrB, trB, semB)
            drain(hrA, trA, semA)
            _compute_block(hrA, trA, relt_v, w_v, ridx, confv, impv, pcav,
                           evv, b * B)
            issue(b + 2, hrA, trA, semA)
            drain(hrB, trB, semB)
            _compute_block(hrB, trB, relt_v, w_v, ridx, confv, impv, pcav,
                           evv, (b + 1) * B)
            return c

        lax.fori_loop(0, (BPS - 1) // 2, pair, 0)
        drain(hrA, trA, semA)
        _compute_block(hrA, trA, relt_v, w_v, ridx, confv, impv, pcav,
                       evv, (BPS - 1) * B)
        pltpu.sync_copy(evv, out_hbm.at[pl.ds(sbbase, SB)])
        return carry

    lax.fori_loop(0, NSB, superblock, 0)


def _sc_body(ent_sr, ent_tg, rel_sr, rel_tg, w_sr, w_tg,
             head_sr, tail_sr, rid_sr, head_tg, tail_tg, rid_tg,
             conf_sr, imp_sr, pca_sr, conf_tg, imp_tg, pca_tg,
             out_sr, out_tg,
             relt_v, w_v, hidx, tidx, ridx, hrA, trA, hrB, trB,
             confv, impv, pcav, evv, semA, semB):
    wid = lax.axis_index("s") * NC + lax.axis_index("c")
    base0 = wid * EPW
    _side_phase(ent_sr, rel_sr, w_sr, head_sr, tail_sr, rid_sr,
                conf_sr, imp_sr, pca_sr, out_sr,
                relt_v, w_v, hidx, tidx, ridx, hrA, trA, hrB, trB,
                confv, impv, pcav, evv, semA, semB, base0)
    _side_phase(ent_tg, rel_tg, w_tg, head_tg, tail_tg, rid_tg,
                conf_tg, imp_tg, pca_tg, out_tg,
                relt_v, w_v, hidx, tidx, ridx, hrA, trA, hrB, trB,
                confv, impv, pcav, evv, semA, semB, base0)


_edge_values = functools.partial(
    pl.kernel,
    out_type=[jax.ShapeDtypeStruct((E,), jnp.float32),
              jax.ShapeDtypeStruct((E,), jnp.float32)],
    mesh=plsc.VectorSubcoreMesh(core_axis_name="c", subcore_axis_name="s"),
    compiler_params=pltpu.CompilerParams(needs_layout_passes=False),
    scratch_types=[
        pltpu.VMEM((R, D), jnp.float32),
        pltpu.VMEM((R,), jnp.float32),
        pltpu.VMEM((SB,), jnp.int32),
        pltpu.VMEM((SB,), jnp.int32),
        pltpu.VMEM((SB,), jnp.int32),
        pltpu.VMEM((B, D), jnp.float32),
        pltpu.VMEM((B, D), jnp.float32),
        pltpu.VMEM((B, D), jnp.float32),
        pltpu.VMEM((B, D), jnp.float32),
        pltpu.VMEM((SB,), jnp.float32),
        pltpu.VMEM((SB,), jnp.float32),
        pltpu.VMEM((SB,), jnp.float32),
        pltpu.VMEM((SB,), jnp.float32),
        pltpu.SemaphoreType.DMA,
        pltpu.SemaphoreType.DMA,
    ],
)(_sc_body)


def kernel(ent_emb_sr, ent_emb_tg, rel_emb_sr, rel_emb_tg,
           head_sr, tail_sr, relation_sr, head_tg, tail_tg, relation_tg,
           rel_conf_sr, rel_imp_sr, pca_sr, rel_conf_tg, rel_imp_tg, pca_tg):
    w_sr, w_tg = _relation_weights(rel_emb_sr, rel_emb_tg)
    i32 = jnp.int32
    ev_sr, ev_tg = _edge_values(
        ent_emb_sr, ent_emb_tg, rel_emb_sr, rel_emb_tg, w_sr, w_tg,
        head_sr.astype(i32), tail_sr.astype(i32), relation_sr.astype(i32),
        head_tg.astype(i32), tail_tg.astype(i32), relation_tg.astype(i32),
        rel_conf_sr, rel_imp_sr, pca_sr, rel_conf_tg, rel_imp_tg, pca_tg)
    ones = jnp.ones((N,), dtype=jnp.float32)
    return (jnp.concatenate([ev_sr, ones]),
            jnp.concatenate([ev_tg, ones]))


# speedup vs baseline: 24.8901x; 1.1501x over previous
import functools
import math

import jax
import jax.numpy as jnp
from jax import lax
from jax.experimental import pallas as pl
from jax.experimental.pallas import tpu as pltpu
from jax.experimental.pallas import tpu_sc as plsc

N = 10000
R = 500
E = 320000
D = 128

NC = 2
NS = 16
NW = NC * NS
EPW = E // NW
B = 80
G = B // 16
SB = 2000
NSB = EPW // SB
BPS = SB // B

DW = D // 2

INV3SQD = 1.0 / (3.0 * math.sqrt(D))


def _pack_bf16(x):
    y = x.astype(jnp.bfloat16).reshape(x.shape[0], DW, 2)
    return lax.bitcast_convert_type(y, jnp.int32)


def _relw_body(rs_ref, rt_ref, wsr_ref, wtg_ref):
    a = rs_ref[...]
    b = rt_ref[...]
    an = a / (jnp.sqrt(jnp.sum(a * a, axis=1, keepdims=True)) + 1e-8)
    bn = b / (jnp.sqrt(jnp.sum(b * b, axis=1, keepdims=True)) + 1e-8)
    sim = lax.dot_general(an, bn, (((1,), (1,)), ((), ())),
                          preferred_element_type=jnp.float32)
    wsr_ref[...] = jnp.max(sim, axis=1)
    wtg_ref[...] = jnp.max(sim, axis=0)


def _relation_weights(rel_emb_sr, rel_emb_tg):
    return pl.pallas_call(
        _relw_body,
        out_shape=[jax.ShapeDtypeStruct((R,), jnp.float32),
                   jax.ShapeDtypeStruct((R,), jnp.float32)],
    )(rel_emb_sr, rel_emb_tg)


def _compute_block(hrows, trows, relt_v, w_v, ridx, confv, impv, pcav,
                   evv, boff):
    for g in range(G):
        lane = lax.iota(jnp.int32, 16)
        rows = lane + (g * 16)
        off = boff + g * 16
        rids = ridx[pl.ds(off, 16)]

        def dbody(dd, accs):
            acc0, acc1 = accs
            col = (lane + dd) & (DW - 1)
            hw = plsc.load_gather(hrows, [rows, col])
            tw = plsc.load_gather(trows, [rows, col])
            rw = plsc.load_gather(relt_v, [rids, col])
            hb = plsc.bitcast(hw, jnp.bfloat16)
            tb = plsc.bitcast(tw, jnp.bfloat16)
            rb = plsc.bitcast(rw, jnp.bfloat16)
            diff = (hb + rb) - tb
            wa = plsc.bitcast(diff, jnp.int32) & jnp.int32(0x7FFF7FFF)
            lo = plsc.bitcast(wa << 16, jnp.float32)
            hi = plsc.bitcast(wa & jnp.int32(-65536), jnp.float32)
            return (acc0 + lo, acc1 + hi)

        zero = jnp.zeros((16,), jnp.float32)
        acc0, acc1 = lax.fori_loop(0, DW, dbody, (zero, zero), unroll=8)
        acc = acc0 + acc1
        att = plsc.load_gather(w_v, [rids])
        tv_score = 1.0 - acc * INV3SQD
        cv = confv[pl.ds(off, 16)]
        iv = impv[pl.ds(off, 16)]
        pv = pcav[pl.ds(off, 16)]
        evv[pl.ds(off, 16)] = cv * iv * (0.3 * pv + 0.3 * tv_score + 0.4 * att)


def _side_phase(ent_hbm, rel_hbm, w_hbm, head_hbm, tail_hbm, rid_hbm,
                conf_hbm, imp_hbm, pca_hbm, out_hbm,
                relt_v, w_v, hidx, tidx, ridx,
                hrA, trA, hrB, trB,
                confv, impv, pcav, evv, semA, semB, base0):
    pltpu.sync_copy(rel_hbm, relt_v)
    pltpu.sync_copy(w_hbm, w_v)

    def issue(bi, hr, tr, sem):
        off = bi * B
        pltpu.async_copy(ent_hbm.at[hidx.at[pl.ds(off, B)]], hr, sem)
        pltpu.async_copy(ent_hbm.at[tidx.at[pl.ds(off, B)]], tr, sem)

    def drain(hr, tr, sem):
        pltpu.make_async_copy(ent_hbm.at[hidx.at[pl.ds(0, B)]], hr, sem).wait()
        pltpu.make_async_copy(ent_hbm.at[tidx.at[pl.ds(0, B)]], tr, sem).wait()

    def superblock(sbi, carry):
        sbbase = base0 + sbi * SB
        pltpu.sync_copy(head_hbm.at[pl.ds(sbbase, SB)], hidx)
        pltpu.sync_copy(tail_hbm.at[pl.ds(sbbase, SB)], tidx)
        pltpu.sync_copy(rid_hbm.at[pl.ds(sbbase, SB)], ridx)
        pltpu.sync_copy(conf_hbm.at[pl.ds(sbbase, SB)], confv)
        pltpu.sync_copy(imp_hbm.at[pl.ds(sbbase, SB)], impv)
        pltpu.sync_copy(pca_hbm.at[pl.ds(sbbase, SB)], pcav)
        issue(0, hrA, trA, semA)

        def pair(i, c):
            b = 2 * i
            issue(b + 1, hrB, trB, semB)
            drain(hrA, trA, semA)
            _compute_block(hrA, trA, relt_v, w_v, ridx, confv, impv, pcav,
                           evv, b * B)
            issue(b + 2, hrA, trA, semA)
            drain(hrB, trB, semB)
            _compute_block(hrB, trB, relt_v, w_v, ridx, confv, impv, pcav,
                           evv, (b + 1) * B)
            return c

        lax.fori_loop(0, (BPS - 1) // 2, pair, 0)
        drain(hrA, trA, semA)
        _compute_block(hrA, trA, relt_v, w_v, ridx, confv, impv, pcav,
                       evv, (BPS - 1) * B)
        pltpu.sync_copy(evv, out_hbm.at[pl.ds(sbbase, SB)])
        return carry

    lax.fori_loop(0, NSB, superblock, 0)


def _sc_body(ent_sr, ent_tg, rel_sr, rel_tg, w_sr, w_tg,
             head_sr, tail_sr, rid_sr, head_tg, tail_tg, rid_tg,
             conf_sr, imp_sr, pca_sr, conf_tg, imp_tg, pca_tg,
             out_sr, out_tg,
             relt_v, w_v, hidx, tidx, ridx, hrA, trA, hrB, trB,
             confv, impv, pcav, evv, semA, semB):
    wid = lax.axis_index("s") * NC + lax.axis_index("c")
    base0 = wid * EPW
    _side_phase(ent_sr, rel_sr, w_sr, head_sr, tail_sr, rid_sr,
                conf_sr, imp_sr, pca_sr, out_sr,
                relt_v, w_v, hidx, tidx, ridx, hrA, trA, hrB, trB,
                confv, impv, pcav, evv, semA, semB, base0)
    _side_phase(ent_tg, rel_tg, w_tg, head_tg, tail_tg, rid_tg,
                conf_tg, imp_tg, pca_tg, out_tg,
                relt_v, w_v, hidx, tidx, ridx, hrA, trA, hrB, trB,
                confv, impv, pcav, evv, semA, semB, base0)


_edge_values = functools.partial(
    pl.kernel,
    out_type=[jax.ShapeDtypeStruct((E,), jnp.float32),
              jax.ShapeDtypeStruct((E,), jnp.float32)],
    mesh=plsc.VectorSubcoreMesh(core_axis_name="c", subcore_axis_name="s"),
    compiler_params=pltpu.CompilerParams(needs_layout_passes=False,
                                         use_tc_tiling_on_sc=False),
    scratch_types=[
        pltpu.VMEM((R, DW), jnp.int32),
        pltpu.VMEM((R,), jnp.float32),
        pltpu.VMEM((SB,), jnp.int32),
        pltpu.VMEM((SB,), jnp.int32),
        pltpu.VMEM((SB,), jnp.int32),
        pltpu.VMEM((B, DW), jnp.int32),
        pltpu.VMEM((B, DW), jnp.int32),
        pltpu.VMEM((B, DW), jnp.int32),
        pltpu.VMEM((B, DW), jnp.int32),
        pltpu.VMEM((SB,), jnp.float32),
        pltpu.VMEM((SB,), jnp.float32),
        pltpu.VMEM((SB,), jnp.float32),
        pltpu.VMEM((SB,), jnp.float32),
        pltpu.SemaphoreType.DMA,
        pltpu.SemaphoreType.DMA,
    ],
)(_sc_body)


def kernel(ent_emb_sr, ent_emb_tg, rel_emb_sr, rel_emb_tg,
           head_sr, tail_sr, relation_sr, head_tg, tail_tg, relation_tg,
           rel_conf_sr, rel_imp_sr, pca_sr, rel_conf_tg, rel_imp_tg, pca_tg):
    w_sr, w_tg = _relation_weights(rel_emb_sr, rel_emb_tg)
    i32 = jnp.int32
    ev_sr, ev_tg = _edge_values(
        _pack_bf16(ent_emb_sr), _pack_bf16(ent_emb_tg),
        _pack_bf16(rel_emb_sr), _pack_bf16(rel_emb_tg), w_sr, w_tg,
        head_sr.astype(i32), tail_sr.astype(i32), relation_sr.astype(i32),
        head_tg.astype(i32), tail_tg.astype(i32), relation_tg.astype(i32),
        rel_conf_sr, rel_imp_sr, pca_sr, rel_conf_tg, rel_imp_tg, pca_tg)
    ones = jnp.ones((N,), dtype=jnp.float32)
    return (jnp.concatenate([ev_sr, ones]),
            jnp.concatenate([ev_tg, ones]))
